# Initial kernel scaffold; baseline (speedup 1.0000x reference)
import functools

import jax
import jax.numpy as jnp
from jax.experimental import pallas as pl
from jax.experimental.pallas import tpu as pltpu

N = 1024
V = 100000
PAD = 0
SMOOTH_W = 0.1 / (V - 2)
CONF = 0.9

BR = 256
BV = 2048
NR = N // BR
NV = (V + BV - 1) // BV


def _loss_kernel(tgt_ref, pred_ref, out_ref, m_ref, s_ref, sum_ref, pt_ref, p0_ref):
    r = pl.program_id(0)
    v = pl.program_id(1)

    @pl.when(v == 0)
    def _init():
        m_ref[...] = jnp.full_like(m_ref, -jnp.inf)
        s_ref[...] = jnp.zeros_like(s_ref)
        sum_ref[...] = jnp.zeros_like(sum_ref)
        pt_ref[...] = jnp.zeros_like(pt_ref)
        p0_ref[...] = pred_ref[:, 0:1]

    x = pred_ref[...]
    cols = v * BV + jax.lax.broadcasted_iota(jnp.int32, (BR, BV), 1)
    t = tgt_ref[0, 0, :].reshape(BR, 1)

    x_for_max = jnp.where(cols < V, x, -jnp.inf)

    m_old = m_ref[...]
    m_new = jnp.maximum(m_old, jnp.max(x_for_max, axis=1, keepdims=True))
    s_ref[...] = (s_ref[...] * jnp.exp(m_old - m_new)
                  + jnp.sum(jnp.exp(x_for_max - m_new), axis=1, keepdims=True))
    m_ref[...] = m_new
    sum_ref[...] += jnp.sum(jnp.where(cols < V, x, 0.0), axis=1, keepdims=True)
    pt_ref[...] += jnp.sum(jnp.where(cols == t, x, 0.0), axis=1, keepdims=True)

    @pl.when(v == NV - 1)
    def _finish():
        lse = m_ref[...] + jnp.log(s_ref[...])
        sum_ls = sum_ref[...] - V * lse
        pt_ls = pt_ref[...] - lse
        p0_ls = p0_ref[...] - lse
        row_loss = -(SMOOTH_W * (sum_ls - p0_ls - pt_ls) + CONF * pt_ls)
        row_loss = jnp.where(t == PAD, 0.0, row_loss)
        partial = jnp.sum(row_loss) / N

        @pl.when(r == 0)
        def _():
            out_ref[0, 0] = partial

        @pl.when(r > 0)
        def _():
            out_ref[0, 0] += partial


@jax.jit
def _label_smoothing_loss(pred, target):
    tgt3 = target.astype(jnp.int32).reshape(NR, 1, BR)
    out = pl.pallas_call(
        _loss_kernel,
        grid=(NR, NV),
        in_specs=[
            pl.BlockSpec((1, 1, BR), lambda r, v: (r, 0, 0)),
            pl.BlockSpec((BR, BV), lambda r, v: (r, v)),
        ],
        out_specs=pl.BlockSpec((1, 1), lambda r, v: (0, 0)),
        out_shape=jax.ShapeDtypeStruct((1, 1), jnp.float32),
        scratch_shapes=[
            pltpu.VMEM((BR, 1), jnp.float32),
            pltpu.VMEM((BR, 1), jnp.float32),
            pltpu.VMEM((BR, 1), jnp.float32),
            pltpu.VMEM((BR, 1), jnp.float32),
            pltpu.VMEM((BR, 1), jnp.float32),
        ],
    )(tgt3, pred)
    return out[0, 0]


def kernel(pred, target):
    return _label_smoothing_loss(pred, target)


# speedup vs baseline: 1.8765x; 1.8765x over previous
import functools

import jax
import jax.numpy as jnp
from jax.experimental import pallas as pl
from jax.experimental.pallas import tpu as pltpu

N = 1024
V = 100000
PAD = 0
SMOOTH_W = 0.1 / (V - 2)
CONF = 0.9

BR = 256
BV = 2048
NR = N // BR
NV = (V + BV - 1) // BV


def _loss_kernel(tgt_ref, pred_ref, out_ref, m_ref, s_ref, sum_ref, pt_ref, p0_ref):
    r = pl.program_id(0)
    v = pl.program_id(1)

    @pl.when(v == 0)
    def _init():
        m_ref[...] = jnp.full_like(m_ref, -jnp.inf)
        s_ref[...] = jnp.zeros_like(s_ref)
        sum_ref[...] = jnp.zeros_like(sum_ref)
        pt_ref[...] = jnp.zeros_like(pt_ref)
        p0_ref[...] = pred_ref[:, 0:1]

    x = pred_ref[...]
    cols = v * BV + jax.lax.broadcasted_iota(jnp.int32, (BR, BV), 1)
    t = tgt_ref[0, 0, :].reshape(BR, 1)

    x_for_max = jnp.where(cols < V, x, -jnp.inf)

    m_old = m_ref[...]
    m_new = jnp.maximum(m_old, jnp.max(x_for_max, axis=1, keepdims=True))
    s_ref[...] = (s_ref[...] * jnp.exp(m_old - m_new)
                  + jnp.sum(jnp.exp(x_for_max - m_new), axis=1, keepdims=True))
    m_ref[...] = m_new
    sum_ref[...] += jnp.sum(jnp.where(cols < V, x, 0.0), axis=1, keepdims=True)
    pt_ref[...] += jnp.sum(jnp.where(cols == t, x, 0.0), axis=1, keepdims=True)

    @pl.when(v == NV - 1)
    def _finish():
        lse = m_ref[...] + jnp.log(s_ref[...])
        sum_ls = sum_ref[...] - V * lse
        pt_ls = pt_ref[...] - lse
        p0_ls = p0_ref[...] - lse
        row_loss = -(SMOOTH_W * (sum_ls - p0_ls - pt_ls) + CONF * pt_ls)
        row_loss = jnp.where(t == PAD, 0.0, row_loss)
        partial = jnp.sum(row_loss, axis=(0, 1), keepdims=True) / N

        @pl.when(r == 0)
        def _():
            out_ref[...] = partial

        @pl.when(r > 0)
        def _():
            out_ref[...] += partial


@jax.jit
def _label_smoothing_loss(pred, target):
    tgt3 = target.astype(jnp.int32).reshape(NR, 1, BR)
    out = pl.pallas_call(
        _loss_kernel,
        grid=(NR, NV),
        in_specs=[
            pl.BlockSpec((1, 1, BR), lambda r, v: (r, 0, 0)),
            pl.BlockSpec((BR, BV), lambda r, v: (r, v)),
        ],
        out_specs=pl.BlockSpec((1, 1), lambda r, v: (0, 0)),
        out_shape=jax.ShapeDtypeStruct((1, 1), jnp.float32),
        scratch_shapes=[
            pltpu.VMEM((BR, 1), jnp.float32),
            pltpu.VMEM((BR, 1), jnp.float32),
            pltpu.VMEM((BR, 1), jnp.float32),
            pltpu.VMEM((BR, 1), jnp.float32),
            pltpu.VMEM((BR, 1), jnp.float32),
        ],
    )(tgt3, pred)
    return out[0, 0]


def kernel(pred, target):
    return _label_smoothing_loss(pred, target)
